# Initial kernel scaffold; baseline (speedup 1.0000x reference)
import jax
import jax.numpy as jnp
from jax.experimental import pallas as pl


def kernel(x, edge_index, edge_attr, node_mask, subgraphs2nodes, atom_emb, edge_emb, eps, W1, b1, bn1_g, bn1_b, bn1_m, bn1_v, W2, b2, bn2_g, bn2_b, bn2_m, bn2_v):
    raise NotImplementedError("write your pallas kernel here")


# speedup vs baseline: 1.4947x; 1.4947x over previous
import functools

import jax
import jax.numpy as jnp
from jax import lax
from jax.experimental import pallas as pl
from jax.experimental.pallas import tpu as pltpu, tpu_sc as plsc

N = 10000
NP = 10240
E = 160000
EP = 163840
D = 300
DP = 384
G = 128
L = 5
NOUT = 2000
NOUTP = 2048
NS = 16
B = 128
NVR = G // 16


def _make_sc_gather_scatter(n_out: int, with_c: bool, jobs0, jobs1, n_outs):
    rpw = n_out // NS
    assert rpw % 8 == 0 and n_out % NS == 0
    mesh = plsc.VectorSubcoreMesh(core_axis_name="c", subcore_axis_name="s")

    out_type = tuple(jax.ShapeDtypeStruct((n_out, G), jnp.float32)
                     for _ in range(n_outs))
    scratch = [
        pltpu.VMEM_SHARED((n_out, G), jnp.float32),
        pltpu.VMEM((B,), jnp.int32),
        pltpu.VMEM((B,), jnp.int32),
        pltpu.VMEM((B,), jnp.int32),
        pltpu.VMEM((B, G), jnp.float32),
        pltpu.VMEM((B, G), jnp.float32),
        pltpu.SemaphoreType.DMA,
    ]
    nzb = rpw // B
    zrem = rpw % B
    assert zrem == 0

    @functools.partial(pl.kernel, mesh=mesh, out_type=out_type,
                       scratch_types=scratch)
    def body(tbl, ctbl, src0, src1, src2, dst, cid0, cid1, cid2, *rest):
        outs = rest[:n_outs]
        agg, src_v, dst_v, cid_v, hrows, crows, sem = rest[n_outs:]
        c = lax.axis_index("c")
        s = lax.axis_index("s")
        srcs = (src0, src1, src2)
        cids = (cid0, cid1, cid2)

        zero16 = jnp.zeros((16,), jnp.float32)

        def zrow(r, carry):
            for j in range(NVR):
                crows[r, pl.ds(j * 16, 16)] = zero16
            return carry

        lax.fori_loop(0, B, zrow, 0)

        def zero_agg():
            for k in range(nzb):
                pltpu.sync_copy(crows.at[pl.ds(0, B)],
                                agg.at[pl.ds(s * rpw + k * B, B)])
            plsc.subcore_barrier()

        def run_jobs(jobs):
            first = True
            for (g, ebase, bps, oi) in jobs:
                if not first:
                    zero_agg()
                first = False
                src_g = srcs[g]
                cid_g = cids[g]

                def block(i, carry):
                    base = ebase + s * (bps * B) + i * B
                    pltpu.sync_copy(src_g.at[pl.ds(base, B)], src_v)
                    pltpu.sync_copy(dst.at[pl.ds(base, B)], dst_v)
                    pltpu.async_copy(tbl.at[src_v], hrows, sem).wait()
                    if with_c:
                        pltpu.sync_copy(cid_g.at[pl.ds(base, B)], cid_v)
                        pltpu.async_copy(ctbl.at[cid_v], crows, sem).wait()

                        def rowbody(r, rc):
                            for j in range(NVR):
                                sl = pl.ds(j * 16, 16)
                                hv = hrows[r, sl]
                                cv = crows[r, sl]
                                hrows[r, sl] = jnp.maximum(hv + cv, 0.0)
                            return rc

                        lax.fori_loop(0, B, rowbody, 0)
                    pltpu.sync_copy(hrows, agg.at[dst_v], add=True)
                    return carry

                lax.fori_loop(0, bps, block, 0)
                plsc.subcore_barrier()
                pltpu.sync_copy(agg.at[pl.ds(s * rpw, rpw)],
                                outs[oi].at[pl.ds(s * rpw, rpw)])
                if with_c:
                    lax.fori_loop(0, B, zrow, 0)

        zero_agg()

        @pl.when(c == 0)
        def _():
            run_jobs(jobs0)

        @pl.when(c == 1)
        def _():
            run_jobs(jobs1)

    return body


def _make_tc_mlp(relu_out: bool, use_mask: bool, split_g2: bool,
                 bn_rows: int):
    grid = (N // bn_rows,)
    n_agg = 4 if split_g2 else 3

    def body(h, *rest):
        aggs = rest[:n_agg]
        w1, b1, w2, b2, epsr = rest[n_agg:n_agg + 5]
        tail = rest[n_agg + 5:]
        if use_mask:
            mask, o = tail
        else:
            (o,) = tail
        if split_g2:
            g2 = aggs[2][...] + aggs[3][...]
        else:
            g2 = aggs[2][...]
        a = jnp.concatenate([aggs[0][...], aggs[1][...], g2], axis=1)
        z = (1.0 + epsr[0, 0]) * h[...] + a
        y = jnp.dot(z, w1[...], preferred_element_type=jnp.float32) + b1[...]
        y = jnp.maximum(y, 0.0)
        z2 = jnp.dot(y, w2[...], preferred_element_type=jnp.float32) + b2[...]
        if relu_out:
            z2 = jnp.maximum(z2, 0.0)
        if use_mask:
            z2 = z2 * mask[...]
        o[...] = z2

    rb = bn_rows
    in_specs = [pl.BlockSpec((rb, DP), lambda i: (i, 0))]
    in_specs += [pl.BlockSpec((rb, G), lambda i: (i, 0))] * n_agg
    in_specs += [
        pl.BlockSpec((DP, 2 * D), lambda i: (0, 0)),
        pl.BlockSpec((1, 2 * D), lambda i: (0, 0)),
        pl.BlockSpec((2 * D, DP), lambda i: (0, 0)),
        pl.BlockSpec((1, DP), lambda i: (0, 0)),
        pl.BlockSpec((1, 1), lambda i: (0, 0)),
    ]
    if use_mask:
        in_specs.append(pl.BlockSpec((rb, 1), lambda i: (i, 0)))
    return pl.pallas_call(
        body,
        grid=grid,
        in_specs=in_specs,
        out_specs=pl.BlockSpec((rb, DP), lambda i: (i, 0)),
        out_shape=jax.ShapeDtypeStruct((N, DP), jnp.float32),
    )


def kernel(x, edge_index, edge_attr, node_mask, subgraphs2nodes, atom_emb,
           edge_emb, eps, W1, b1, bn1_g, bn1_b, bn1_m, bn1_v,
           W2, b2, bn2_g, bn2_b, bn2_m, bn2_v):
    f32 = jnp.float32
    i32 = jnp.int32

    src = jnp.pad(edge_index[0].astype(i32), (0, EP - E), constant_values=0)
    dst = jnp.pad(edge_index[1].astype(i32), (0, EP - E),
                  constant_values=NP - 1)
    srcg = tuple(src * 3 + g for g in range(3))
    ea = edge_attr.astype(i32)
    cid = jnp.pad(ea[:, 0] * 25 + ea[:, 1] * 5 + ea[:, 2], (0, EP - E),
                  constant_values=125)
    cidg = tuple(cid * 3 + g for g in range(3))

    ax = (x.astype(i32) + jnp.arange(9, dtype=i32)[None, :] * 100)
    AE = 9 * N
    AEP = 16 * 44 * B
    a_src = jnp.pad(ax.reshape(-1), (0, AEP - AE), constant_values=900)
    a_srcg = tuple(a_src * 3 + g for g in range(3))
    a_dst = jnp.pad(jnp.repeat(jnp.arange(N, dtype=i32), 9), (0, AEP - AE),
                    constant_values=NP - 1)

    atab = atom_emb.reshape(900, D).astype(f32)
    atab = jnp.pad(atab, ((0, 4), (0, DP - D))).reshape(904 * 3, G)

    ee = edge_emb.astype(f32)
    C = (ee[:, 0][:, :, None, None, :] + ee[:, 1][:, None, :, None, :]
         + ee[:, 2][:, None, None, :, :]).reshape(L, 125, D)
    C = jnp.pad(C, ((0, 0), (0, 3), (0, DP - D)))
    C3 = C.reshape(L, 128 * 3, G)

    s1 = bn1_g / jnp.sqrt(bn1_v + 1e-5)
    W1f = W1 * s1[:, None, :]
    b1f = (b1 - bn1_m) * s1 + bn1_b
    W1p = jnp.pad(W1f, ((0, 0), (0, DP - D), (0, 0)))
    s2 = bn2_g / jnp.sqrt(bn2_v + 1e-5)
    W2f = W2 * s2[:, None, :]
    b2f = (b2 - bn2_m) * s2 + bn2_b
    W2p = jnp.pad(W2f, ((0, 0), (0, 0), (0, DP - D)))
    b2p = jnp.pad(b2f, ((0, 0), (0, DP - D)))

    maskc = node_mask.astype(f32).reshape(N, 1)
    dummyc = jnp.zeros((8, G), f32)
    dummyi = jnp.zeros((B,), i32)

    sc_msg = _make_sc_gather_scatter(
        NP, True,
        jobs0=[(0, 0, 80, 0), (2, 0, 40, 2)],
        jobs1=[(1, 0, 80, 1), (2, EP // 2, 40, 3)],
        n_outs=4)
    sc_enc = _make_sc_gather_scatter(
        NP, False,
        jobs0=[(0, 0, 44, 0), (2, 0, 44, 2)],
        jobs1=[(1, 0, 44, 1)],
        n_outs=3)
    sc_pool = _make_sc_gather_scatter(
        NOUTP, False,
        jobs0=[(0, 0, 5, 0), (2, 0, 5, 2)],
        jobs1=[(1, 0, 5, 1)],
        n_outs=3)
    tc_mid = _make_tc_mlp(True, False, True, bn_rows=1000)
    tc_last = _make_tc_mlp(False, True, True, bn_rows=1000)

    e0, e1, e2 = sc_enc(atab, dummyc, a_srcg[0], a_srcg[1], a_srcg[2],
                        a_dst, dummyi, dummyi, dummyi)
    h = jnp.stack([e0[:N], e1[:N], e2[:N]], axis=1).reshape(3 * N, G)

    for l in range(L):
        g0, g1, g2a, g2b = sc_msg(h, C3[l], srcg[0], srcg[1], srcg[2],
                                  dst, cidg[0], cidg[1], cidg[2])
        epsv = eps[l].astype(f32).reshape(1, 1)
        hm = h.reshape(N, DP)
        if l < L - 1:
            hm = tc_mid(hm, g0, g1, g2a, g2b, W1p[l], b1f[l][None, :],
                        W2p[l], b2p[l][None, :], epsv)
        else:
            hm = tc_last(hm, g0, g1, g2a, g2b, W1p[l], b1f[l][None, :],
                         W2p[l], b2p[l][None, :], epsv, maskc)
        h = hm.reshape(3 * N, G)

    hp = jnp.pad(h.reshape(N, DP), ((0, NP - N), (0, 0))).reshape(3 * NP, G)
    p_src = jnp.arange(NP, dtype=i32)
    p_srcg = tuple(p_src * 3 + g for g in range(3))
    p_dst = jnp.pad(subgraphs2nodes.astype(i32), (0, NP - N),
                    constant_values=NOUTP - 1)
    o0, o1, o2 = sc_pool(hp, dummyc, p_srcg[0], p_srcg[1], p_srcg[2],
                         p_dst, dummyi, dummyi, dummyi)

    return jnp.concatenate([o0[:NOUT], o1[:NOUT], o2[:NOUT, :D - 2 * G]],
                           axis=1)


# speedup vs baseline: 1.5618x; 1.0449x over previous
import functools

import jax
import jax.numpy as jnp
from jax import lax
from jax.experimental import pallas as pl
from jax.experimental.pallas import tpu as pltpu, tpu_sc as plsc

N = 10000
NP = 10112
E = 160000
EP = 163840
D = 300
DP = 384
G = 128
L = 5
NOUT = 2000
NOUTP = 2048
NS = 16
B = 128
NVR = G // 16
AEP = 16 * 48 * B
PEP = 16 * 8 * B


def _make_sc_gather_scatter(n_out: int, with_c: bool, jobs0, jobs1, n_outs):
    rpw = n_out // NS
    assert rpw % 8 == 0
    mesh = plsc.VectorSubcoreMesh(core_axis_name="c", subcore_axis_name="s")

    out_type = tuple(jax.ShapeDtypeStruct((n_out, G), jnp.float32)
                     for _ in range(n_outs))
    scratch = [
        pltpu.VMEM_SHARED((n_out, G), jnp.float32),
        pltpu.VMEM((B, G), jnp.float32),
        pltpu.VMEM((B, G), jnp.float32),
        pltpu.VMEM((B, G), jnp.float32),
        pltpu.VMEM((2 * B,), jnp.int32),
        pltpu.VMEM((B,), jnp.int32),
        pltpu.VMEM((B,), jnp.int32),
        pltpu.VMEM((B,), jnp.int32),
        pltpu.VMEM((B,), jnp.int32),
        pltpu.SemaphoreType.DMA,
        pltpu.SemaphoreType.DMA,
        pltpu.SemaphoreType.DMA,
        pltpu.SemaphoreType.DMA,
        pltpu.SemaphoreType.DMA,
    ]

    @functools.partial(pl.kernel, mesh=mesh, out_type=out_type,
                       scratch_types=scratch)
    def body(tbl, ctbl, src0, src1, src2, dst, cid0, cid1, cid2, *rest):
        outs = rest[:n_outs]
        (agg, h0b, h1b, ccb, src2v, cidv0, cidv1, dst0v, dst1v,
         semh0, semh1, semc, sems0, sems1) = rest[n_outs:]
        c = lax.axis_index("c")
        s = lax.axis_index("s")
        srcs = (src0, src1, src2)
        cids = (cid0, cid1, cid2)
        hb = (h0b, h1b)
        cv = (cidv0, cidv1)
        dv = (dst0v, dst1v)
        semh = (semh0, semh1)
        sems = (sems0, sems1)

        zero16 = jnp.zeros((16,), jnp.float32)

        def zrow(r, carry):
            for j in range(NVR):
                ccb[r, pl.ds(j * 16, 16)] = zero16
            return carry

        def zero_agg():
            full, rem = divmod(rpw, B)
            for k in range(full):
                pltpu.sync_copy(ccb, agg.at[pl.ds(s * rpw + k * B, B)])
            if rem:
                pltpu.sync_copy(ccb.at[pl.ds(0, rem)],
                                agg.at[pl.ds(s * rpw + full * B, rem)])
            plsc.subcore_barrier()

        def combine(hbuf):
            def rowbody(r, rc):
                for j in range(NVR):
                    sl = pl.ds(j * 16, 16)
                    hbuf[r, sl] = jnp.maximum(hbuf[r, sl] + ccb[r, sl], 0.0)
                return rc

            lax.fori_loop(0, B, rowbody, 0)

        lax.fori_loop(0, B, zrow, 0)

        def run_jobs(jobs):
            first = True
            for (g, ebase, bps, oi) in jobs:
                if not first:
                    if with_c:
                        lax.fori_loop(0, B, zrow, 0)
                    zero_agg()
                first = False
                src_g = srcs[g]
                cid_g = cids[g]

                def do_pair(k, carry):
                    base = ebase + s * (bps * B) + 2 * k * B
                    pltpu.sync_copy(src_g.at[pl.ds(base, 2 * B)], src2v)
                    pltpu.sync_copy(dst.at[pl.ds(base, B)], dst0v)
                    pltpu.sync_copy(dst.at[pl.ds(base + B, B)], dst1v)
                    if with_c:
                        pltpu.sync_copy(cid_g.at[pl.ds(base, B)], cidv0)
                        pltpu.sync_copy(cid_g.at[pl.ds(base + B, B)], cidv1)
                    hg = [pltpu.async_copy(
                        tbl.at[src2v.at[pl.ds(p * B, B)]], hb[p], semh[p])
                        for p in (0, 1)]
                    scs = []
                    for p in (0, 1):
                        if with_c:
                            cg = pltpu.async_copy(
                                ctbl.at[cv[p]], ccb, semc)
                        hg[p].wait()
                        if with_c:
                            cg.wait()
                            combine(hb[p])
                        scs.append(pltpu.async_copy(
                            hb[p], agg.at[dv[p]], sems[p], add=True))
                    for sc in scs:
                        sc.wait()
                    return carry

                lax.fori_loop(0, bps // 2, do_pair, 0)
                plsc.subcore_barrier()
                pltpu.sync_copy(agg.at[pl.ds(s * rpw, rpw)],
                                outs[oi].at[pl.ds(s * rpw, rpw)])

        zero_agg()

        @pl.when(c == 0)
        def _():
            run_jobs(jobs0)

        @pl.when(c == 1)
        def _():
            run_jobs(jobs1)

    return body


def _make_tc_mlp(relu_out: bool, use_mask: bool, split_g2: bool,
                 bn_rows: int):
    grid = (N // bn_rows,)
    n_agg = 4 if split_g2 else 3

    def body(h, *rest):
        aggs = rest[:n_agg]
        w1, b1, w2, b2, epsr = rest[n_agg:n_agg + 5]
        tail = rest[n_agg + 5:]
        if use_mask:
            mask, o = tail
        else:
            (o,) = tail
        if split_g2:
            g2 = aggs[2][...] + aggs[3][...]
        else:
            g2 = aggs[2][...]
        a = jnp.concatenate([aggs[0][...], aggs[1][...], g2], axis=1)
        z = (1.0 + epsr[0, 0]) * h[...] + a
        y = jnp.dot(z, w1[...], preferred_element_type=jnp.float32) + b1[...]
        y = jnp.maximum(y, 0.0)
        z2 = jnp.dot(y, w2[...], preferred_element_type=jnp.float32) + b2[...]
        if relu_out:
            z2 = jnp.maximum(z2, 0.0)
        if use_mask:
            z2 = z2 * mask[...]
        o[...] = z2

    rb = bn_rows
    in_specs = [pl.BlockSpec((rb, DP), lambda i: (i, 0))]
    in_specs += [pl.BlockSpec((rb, G), lambda i: (i, 0))] * n_agg
    in_specs += [
        pl.BlockSpec((DP, 2 * D), lambda i: (0, 0)),
        pl.BlockSpec((1, 2 * D), lambda i: (0, 0)),
        pl.BlockSpec((2 * D, DP), lambda i: (0, 0)),
        pl.BlockSpec((1, DP), lambda i: (0, 0)),
        pl.BlockSpec((1, 1), lambda i: (0, 0)),
    ]
    if use_mask:
        in_specs.append(pl.BlockSpec((rb, 1), lambda i: (i, 0)))
    return pl.pallas_call(
        body,
        grid=grid,
        in_specs=in_specs,
        out_specs=pl.BlockSpec((rb, DP), lambda i: (i, 0)),
        out_shape=jax.ShapeDtypeStruct((N, DP), jnp.float32),
    )


def kernel(x, edge_index, edge_attr, node_mask, subgraphs2nodes, atom_emb,
           edge_emb, eps, W1, b1, bn1_g, bn1_b, bn1_m, bn1_v,
           W2, b2, bn2_g, bn2_b, bn2_m, bn2_v):
    f32 = jnp.float32
    i32 = jnp.int32

    src = jnp.pad(edge_index[0].astype(i32), (0, EP - E), constant_values=0)
    dst = jnp.pad(edge_index[1].astype(i32), (0, EP - E),
                  constant_values=NP - 1)
    srcg = tuple(src * 3 + g for g in range(3))
    ea = edge_attr.astype(i32)
    cid = jnp.pad(ea[:, 0] * 25 + ea[:, 1] * 5 + ea[:, 2], (0, EP - E),
                  constant_values=125)
    cidg = tuple(cid * 3 + g for g in range(3))

    ax = (x.astype(i32) + jnp.arange(9, dtype=i32)[None, :] * 100)
    a_src = jnp.pad(ax.reshape(-1), (0, AEP - 9 * N), constant_values=900)
    a_srcg = tuple(a_src * 3 + g for g in range(3))
    a_dst = jnp.pad(jnp.repeat(jnp.arange(N, dtype=i32), 9),
                    (0, AEP - 9 * N), constant_values=NP - 1)

    atab = atom_emb.reshape(900, D).astype(f32)
    atab = jnp.pad(atab, ((0, 4), (0, DP - D))).reshape(904 * 3, G)

    ee = edge_emb.astype(f32)
    C = (ee[:, 0][:, :, None, None, :] + ee[:, 1][:, None, :, None, :]
         + ee[:, 2][:, None, None, :, :]).reshape(L, 125, D)
    C = jnp.pad(C, ((0, 0), (0, 3), (0, DP - D)))
    C3 = C.reshape(L, 128 * 3, G)

    s1 = bn1_g / jnp.sqrt(bn1_v + 1e-5)
    W1f = W1 * s1[:, None, :]
    b1f = (b1 - bn1_m) * s1 + bn1_b
    W1p = jnp.pad(W1f, ((0, 0), (0, DP - D), (0, 0)))
    s2 = bn2_g / jnp.sqrt(bn2_v + 1e-5)
    W2f = W2 * s2[:, None, :]
    b2f = (b2 - bn2_m) * s2 + bn2_b
    W2p = jnp.pad(W2f, ((0, 0), (0, 0), (0, DP - D)))
    b2p = jnp.pad(b2f, ((0, 0), (0, DP - D)))

    maskc = node_mask.astype(f32).reshape(N, 1)
    dummyc = jnp.zeros((G, G), f32)
    dummyi = jnp.zeros((B,), i32)

    sc_msg = _make_sc_gather_scatter(
        NP, True,
        jobs0=[(0, 0, 80, 0), (2, 0, 40, 2)],
        jobs1=[(1, 0, 80, 1), (2, EP // 2, 40, 3)],
        n_outs=4)
    sc_enc = _make_sc_gather_scatter(
        NP, False,
        jobs0=[(0, 0, 48, 0), (2, 0, 48, 2)],
        jobs1=[(1, 0, 48, 1)],
        n_outs=3)
    sc_pool = _make_sc_gather_scatter(
        NOUTP, False,
        jobs0=[(0, 0, 8, 0), (2, 0, 8, 2)],
        jobs1=[(1, 0, 8, 1)],
        n_outs=3)
    tc_mid = _make_tc_mlp(True, False, True, bn_rows=1000)
    tc_last = _make_tc_mlp(False, True, True, bn_rows=1000)

    e0, e1, e2 = sc_enc(atab, dummyc, a_srcg[0], a_srcg[1],
                        a_srcg[2], a_dst, dummyi, dummyi, dummyi)
    h = jnp.stack([e0[:N], e1[:N], e2[:N]], axis=1).reshape(3 * N, G)

    for l in range(L):
        g0, g1, g2a, g2b = sc_msg(h, C3[l], srcg[0], srcg[1], srcg[2],
                                  dst, cidg[0], cidg[1], cidg[2])
        epsv = eps[l].astype(f32).reshape(1, 1)
        hm = h.reshape(N, DP)
        if l < L - 1:
            hm = tc_mid(hm, g0, g1, g2a, g2b, W1p[l], b1f[l][None, :],
                        W2p[l], b2p[l][None, :], epsv)
        else:
            hm = tc_last(hm, g0, g1, g2a, g2b, W1p[l], b1f[l][None, :],
                         W2p[l], b2p[l][None, :], epsv, maskc)
        h = hm.reshape(3 * N, G)

    NPAD = N + 8
    hp = jnp.pad(h.reshape(N, DP), ((0, NPAD - N), (0, 0)))
    hp = hp.reshape(3 * NPAD, G)
    p_src = jnp.pad(jnp.arange(N, dtype=i32), (0, PEP - N),
                    constant_values=N)
    p_srcg = tuple(p_src * 3 + g for g in range(3))
    p_dst = jnp.pad(subgraphs2nodes.astype(i32), (0, PEP - N),
                    constant_values=NOUTP - 1)
    o0, o1, o2 = sc_pool(hp, dummyc, p_srcg[0], p_srcg[1],
                         p_srcg[2], p_dst, dummyi, dummyi, dummyi)

    return jnp.concatenate([o0[:NOUT], o1[:NOUT], o2[:NOUT, :D - 2 * G]],
                           axis=1)
